# baseline (speedup 1.0000x reference)
import functools

import jax
import jax.numpy as jnp
import numpy as np
from jax import lax
from jax.experimental import pallas as pl
from jax.experimental.pallas import tpu as pltpu
from jax.experimental.pallas import tpu_sc as plsc

_B = 4096
_S = 200
_D = 64
_NW = 32
_RPW = _B // _NW
_NV = 13
_PAD = np.uint32(0xFFFFFFFF)

def _vsort(v):
    if v is None:
        return None
    return lax.sort(v, dimension=0)


def _vrev(v):
    if v is None:
        return None
    return lax.rev(v, (0,))


def _ce(a, b):
    if a is None and b is None:
        return None, None
    if a is None:
        return b, None
    if b is None:
        return a, None
    return jnp.minimum(a, b), jnp.maximum(a, b)


def _bitonic_merge(c):
    c = list(c)
    m = len(c)
    d = m // 2
    while d >= 1:
        for i in range(m):
            if (i % (2 * d)) < d:
                c[i], c[i + d] = _ce(c[i], c[i + d])
        d //= 2
    return [_vsort(v) for v in c]


def _merge_runs(a, b):
    return _bitonic_merge(list(a) + [_vrev(v) for v in reversed(b)])


def _sort_vecs(vecs):
    runs = [[_vsort(v)] for v in vecs]
    while len(runs) > 1:
        runs = [_merge_runs(runs[2 * i], runs[2 * i + 1])
                for i in range(len(runs) // 2)]
    return runs[0]


def _make_sc_call_b():
    mesh = plsc.VectorSubcoreMesh(core_axis_name="c", subcore_axis_name="s",
                                  num_cores=2, num_subcores=16)

    @functools.partial(
        pl.kernel,
        mesh=mesh,
        out_type=jax.ShapeDtypeStruct((_B, _S, _D), jnp.float32),
        compiler_params=pltpu.CompilerParams(needs_layout_passes=False),
        scratch_types=[
            pltpu.VMEM((_RPW * _S + 16,), jnp.float32),
            pltpu.VMEM((_S * _D,), jnp.float32),
            pltpu.VMEM((_S, _D), jnp.float32),
            pltpu.VMEM((_S, _D), jnp.float32),
            pltpu.SemaphoreType.DMA,
            pltpu.SemaphoreType.DMA,
        ],
    )
    def sc_kernel(ts_hbm, table_hbm, out_hbm, ts_v, table_v, rb0, rb1,
                  sem0, sem1):
        wid = lax.axis_index("c") * 16 + lax.axis_index("s")
        pltpu.sync_copy(ts_hbm.at[pl.ds(wid * (_RPW * _S), _RPW * _S)],
                        ts_v.at[pl.ds(0, _RPW * _S)])
        pltpu.sync_copy(table_hbm, table_v)

        lane = jnp.arange(16, dtype=jnp.int32)
        lane_u = lane.astype(jnp.uint32)
        tail_mask = lane < 8
        dq = [16 * q + lane for q in range(4)]
        slots = ((rb0, sem0), (rb1, sem1))

        def do_row(r, rb, sem):
            roff = r * _S
            row_abs = wid * _RPW + r

            keys = []
            for vi in range(_NV):
                kv = ts_v[pl.ds(roff + 16 * vi, 16)]
                u = (kv * jnp.float32(8388608.0)).astype(jnp.uint32)
                kk = (u << jnp.uint32(9)) | (lane_u + jnp.uint32(16 * vi))
                if vi == _NV - 1:
                    kk = jnp.where(tail_mask, kk, _PAD)
                keys.append(kk)
            keys += [None] * (16 - _NV)
            s = _sort_vecs(keys)

            for i in range(_NV):
                ord64 = ((s[i] & jnp.uint32(511)).astype(jnp.int32)
                         << jnp.int32(6))
                nl = 8 if i == _NV - 1 else 16
                for l in range(nl):
                    base = jnp.broadcast_to(ord64[l], (16,))
                    pos = 16 * i + l
                    for q in range(4):
                        vals = plsc.load_gather(table_v, [base + dq[q]])
                        rb[pos, pl.ds(16 * q, 16)] = vals

            pltpu.async_copy(rb, out_hbm.at[row_abs], sem)

        def drain(rb, sem):
            pltpu.make_async_copy(rb, out_hbm.at[0], sem).wait()

        def iter_body(g, carry):
            for half, (rb, sem) in enumerate(slots):
                @pl.when(g > 0)
                def _():
                    drain(rb, sem)
                do_row(2 * g + half, rb, sem)
            return carry

        lax.fori_loop(0, _RPW // 2, iter_body, 0)
        for rb, sem in slots:
            drain(rb, sem)

    return sc_kernel


def kernel(ts, pos_table):
    b_, s_ = ts.shape
    d_ = pos_table.shape[1]
    return _make_sc_call_b()(ts.reshape(b_ * s_), pos_table.reshape(s_ * d_))


def kernel(ts, pos_table):
    b_, s_ = ts.shape
    d_ = pos_table.shape[1]
    return _make_sc_call_b()(ts.reshape(b_ * s_), pos_table.reshape(s_ * d_))
